# baseline (speedup 1.0000x reference)
import functools

import jax
import jax.numpy as jnp
from jax import lax
from jax.experimental import pallas as pl
from jax.experimental.pallas import tpu as pltpu
from jax.experimental.pallas import tpu_sc as plsc

N_FEATURES = 26
FEATURE_SIZE = 100000
EMB_DIM = 32
BATCH = 16384
TOTAL = BATCH * N_FEATURES
IB = 1024
NTASK = N_FEATURES * (BATCH // IB)

_INFO = plsc.get_sparse_core_info()
NC = _INFO.num_cores
NS = _INFO.num_subcores
NW = NC * NS
TPW = NTASK // NW


@functools.partial(
    pl.kernel,
    mesh=plsc.VectorSubcoreMesh(core_axis_name="c", subcore_axis_name="s"),
    out_type=jax.ShapeDtypeStruct((TOTAL, EMB_DIM), jnp.float32),
    scratch_types=[
        pltpu.VMEM((IB // 128, 128), jnp.int32),
        pltpu.VMEM((IB // 128, 128), jnp.int32),
        pltpu.VMEM((128, EMB_DIM), jnp.float32),
        pltpu.SemaphoreType.DMA,
        pltpu.SemaphoreType.DMA,
    ],
    compiler_params=pltpu.CompilerParams(use_tc_tiling_on_sc=False),
)
def _lookup_kernel(ids_hbm, table_hbm, out_hbm, idx_v, sidx_v, rows_v, gsem, ssem):
    wid = lax.axis_index("s") * NC + lax.axis_index("c")
    iota = lax.iota(jnp.int32, 16)

    def per_task(n, carry):
        task = n * NW + wid
        t = task // (BATCH // IB)
        ib = task % (BATCH // IB)
        pltpu.sync_copy(ids_hbm.at[t, pl.ds(ib * 8, 8)], idx_v)
        off = t * FEATURE_SIZE

        def fix(q, inner):
            for s in range(8):
                sl = pl.ds(s * 16, 16)
                j = (ib * IB + q * 128 + s * 16) + iota
                idx_v[q, sl] = idx_v[q, sl] + off
                sidx_v[q, sl] = j * N_FEATURES + t
            return inner

        lax.fori_loop(0, IB // 128, fix, 0)

        def move(q, inner):
            pltpu.async_copy(table_hbm.at[idx_v.at[q]], rows_v, gsem).wait()
            pltpu.async_copy(rows_v, out_hbm.at[sidx_v.at[q]], ssem).wait()
            return inner

        lax.fori_loop(0, IB // 128, move, 0)
        return carry

    lax.fori_loop(0, TPW, per_task, 0)


def kernel(feature_ids, embed_weight):
    ids3 = feature_ids.T.reshape(N_FEATURES, BATCH // 128, 128)
    out = _lookup_kernel(ids3, embed_weight)
    return out.reshape(BATCH, N_FEATURES, EMB_DIM)
